# Initial kernel scaffold; baseline (speedup 1.0000x reference)
import jax
import jax.numpy as jnp
from jax.experimental import pallas as pl


def kernel(graph_x, node_pos, surface_x, verts, vnormals, vert_nbr_dist, nbr_vids, vert_nbr_ind, W_chem1, b_chem1, W_chem2, b_chem2, W_surf1, b_surf1, W_surf2, b_surf2, W_geom1, b_geom1, W_geom2, b_geom2, W_feat1, b_feat1, W_feat2, b_feat2):
    raise NotImplementedError("write your pallas kernel here")


# speedup vs baseline: 2.6262x; 2.6262x over previous
import functools

import jax
import jax.numpy as jnp
from jax import lax
from jax.experimental import pallas as pl
from jax.experimental.pallas import tpu as pltpu
from jax.experimental.pallas import tpu_sc as plsc

H = 32
N_NODES = 10000
N_VERTS = 50000
E = 320000

NC = 2
NS = 16
NW = NC * NS
EW = E // NW
CK = 1000
NCHUNK = EW // CK

VH = 25088
ACC_ROWS = VH + 16
TROWS = VH // NS
ESUB = E // NS
CKS = 800
NCH_S = ESUB // CKS

BR = 5000


def _bn(y):
    m = jnp.mean(y, axis=0, keepdims=True)
    v = jnp.mean((y - m) * (y - m), axis=0, keepdims=True)
    return (y - m) * lax.rsqrt(v + 1e-5)


def _gdf(x, start, stop):
    step = (stop - start) / 7.0
    c = start + step * lax.broadcasted_iota(jnp.int32, (1, 8), 1).astype(jnp.float32)
    d = x - c
    return jnp.exp(-(d * d) / (step * step))


def _silu(x):
    return x * jax.nn.sigmoid(x)


def _chem_body(gx_ref, wc1, bc1, wc2, bc2, wp, chem_ref, tp_ref):
    gx = gx_ref[...]
    h = _silu(_bn(jnp.dot(gx, wc1[...], preferred_element_type=jnp.float32) + bc1[...]))
    chem_ref[...] = _bn(jnp.dot(h, wc2[...], preferred_element_type=jnp.float32) + bc2[...])
    tp_ref[...] = jnp.dot(gx, wp[...], preferred_element_type=jnp.float32)


def _chem_stage(graph_x, wc1, bc1, wc2, bc2, wp):
    return pl.pallas_call(
        _chem_body,
        out_shape=(
            jax.ShapeDtypeStruct((N_NODES, H), jnp.float32),
            jax.ShapeDtypeStruct((N_NODES, H), jnp.float32),
        ),
    )(graph_x, wc1, bc1, wc2, bc2, wp)


def _lin1_body(x_ref, w1, b1, y_ref, st_ref):
    i = pl.program_id(0)
    y = jnp.dot(x_ref[...], w1[...], preferred_element_type=jnp.float32) + b1[...]
    y_ref[...] = y

    @pl.when(i == 0)
    def _():
        st_ref[...] = jnp.zeros_like(st_ref)

    st_ref[0:1, :] += jnp.sum(y, axis=0, keepdims=True)
    st_ref[1:2, :] += jnp.sum(y * y, axis=0, keepdims=True)


def _lin1_stage(x, w1, b1):
    n, fi = x.shape
    return pl.pallas_call(
        _lin1_body,
        grid=(n // BR,),
        in_specs=[
            pl.BlockSpec((BR, fi), lambda i: (i, 0)),
            pl.BlockSpec((fi, H), lambda i: (0, 0)),
            pl.BlockSpec((1, H), lambda i: (0, 0)),
        ],
        out_specs=[
            pl.BlockSpec((BR, H), lambda i: (i, 0)),
            pl.BlockSpec((8, H), lambda i: (0, 0)),
        ],
        out_shape=(
            jax.ShapeDtypeStruct((n, H), jnp.float32),
            jax.ShapeDtypeStruct((8, H), jnp.float32),
        ),
    )(x, w1, b1)


def _mid_body(y_ref, mu1, is1, w2, b2, st_ref):
    i = pl.program_id(0)
    h = _silu((y_ref[...] - mu1[...]) * is1[...])
    z = jnp.dot(h, w2[...], preferred_element_type=jnp.float32) + b2[...]

    @pl.when(i == 0)
    def _():
        st_ref[...] = jnp.zeros_like(st_ref)

    st_ref[0:1, :] += jnp.sum(z, axis=0, keepdims=True)
    st_ref[1:2, :] += jnp.sum(z * z, axis=0, keepdims=True)


def _mid_stage(y, mu1, is1, w2, b2):
    n, _ = y.shape
    wo = w2.shape[1]
    return pl.pallas_call(
        _mid_body,
        grid=(n // BR,),
        in_specs=[
            pl.BlockSpec((BR, H), lambda i: (i, 0)),
            pl.BlockSpec((1, H), lambda i: (0, 0)),
            pl.BlockSpec((1, H), lambda i: (0, 0)),
            pl.BlockSpec((H, wo), lambda i: (0, 0)),
            pl.BlockSpec((1, wo), lambda i: (0, 0)),
        ],
        out_specs=pl.BlockSpec((8, wo), lambda i: (0, 0)),
        out_shape=jax.ShapeDtypeStruct((8, wo), jnp.float32),
    )(y, mu1, is1, w2, b2)


def _apply_body(y_ref, mu1, is1, w2s, b2s, out_ref):
    h = _silu((y_ref[...] - mu1[...]) * is1[...])
    out_ref[...] = jnp.dot(h, w2s[...], preferred_element_type=jnp.float32) + b2s[...]


def _apply_stage(y, mu1, is1, w2s, b2s):
    n, _ = y.shape
    return pl.pallas_call(
        _apply_body,
        grid=(n // BR,),
        in_specs=[
            pl.BlockSpec((BR, H), lambda i: (i, 0)),
            pl.BlockSpec((1, H), lambda i: (0, 0)),
            pl.BlockSpec((1, H), lambda i: (0, 0)),
            pl.BlockSpec((H, H), lambda i: (0, 0)),
            pl.BlockSpec((1, H), lambda i: (0, 0)),
        ],
        out_specs=pl.BlockSpec((BR, H), lambda i: (i, 0)),
        out_shape=jax.ShapeDtypeStruct((n, H), jnp.float32),
    )(y, mu1, is1, w2s, b2s)


def _moments(st, n):
    mu = st[0] / n
    var = st[1] / n - mu * mu
    return mu.reshape(1, -1), lax.rsqrt(var + 1e-5).reshape(1, -1)


def _fold_bn2(w2, b2, mu2, is2):
    return w2 * is2, (b2 - mu2) * is2


def _gather_sc(tp, tnp, tvv, ind, vids):
    mesh = plsc.VectorSubcoreMesh(core_axis_name="c", subcore_axis_name="s")

    @functools.partial(
        pl.kernel,
        out_type=(
            jax.ShapeDtypeStruct((E, H), jnp.float32),
            jax.ShapeDtypeStruct((E, 16), jnp.float32),
            jax.ShapeDtypeStruct((E, 16), jnp.float32),
        ),
        mesh=mesh,
        scratch_types=[
            pltpu.VMEM((CK,), jnp.int32),
            pltpu.VMEM((CK,), jnp.int32),
            pltpu.VMEM((CK, H), jnp.float32),
            pltpu.VMEM((CK, 16), jnp.float32),
            pltpu.VMEM((CK, 16), jnp.float32),
            pltpu.SemaphoreType.DMA,
            pltpu.SemaphoreType.DMA,
            pltpu.SemaphoreType.DMA,
        ],
        compiler_params=pltpu.CompilerParams(use_tc_tiling_on_sc=False),
    )
    def k(tp_hbm, tnp_hbm, tvv_hbm, ind_hbm, vids_hbm, oa_hbm, ob_hbm, oc_hbm,
          idx1_v, idx2_v, ra_v, rb_v, rc_v, sem_a, sem_b, sem_c):
        wid = lax.axis_index("s") * NC + lax.axis_index("c")
        for i in range(NCHUNK):
            base = wid * EW + i * CK
            pltpu.sync_copy(ind_hbm.at[pl.ds(base, CK)], idx1_v)
            pltpu.sync_copy(vids_hbm.at[pl.ds(base, CK)], idx2_v)
            da = pltpu.async_copy(tp_hbm.at[idx1_v], ra_v, sem_a)
            db = pltpu.async_copy(tnp_hbm.at[idx1_v], rb_v, sem_b)
            dc = pltpu.async_copy(tvv_hbm.at[idx2_v], rc_v, sem_c)
            da.wait()
            db.wait()
            dc.wait()
            pltpu.sync_copy(ra_v, oa_hbm.at[pl.ds(base, CK)])
            pltpu.sync_copy(rb_v, ob_hbm.at[pl.ds(base, CK)])
            pltpu.sync_copy(rc_v, oc_hbm.at[pl.ds(base, CK)])

    return k(tp, tnp, tvv, ind, vids)


def _edge_a_body(ga_ref, gnp_ref, gvv_ref, d_ref, w1d, w1a, b1, y_ref, st_ref):
    i = pl.program_id(0)
    p8 = gnp_ref[:, 0:8]
    v8 = gvv_ref[:, 0:8]
    n8 = gvv_ref[:, 8:16]
    dlt = p8 - v8
    nrm2 = jnp.sum(dlt * dlt, axis=1, keepdims=True)
    dot = jnp.sum(dlt * n8, axis=1, keepdims=True)
    ang = dot * lax.rsqrt(nrm2)
    gd = _gdf(d_ref[...], 0.0, 8.0)
    ga = _gdf(ang, -1.0, 1.0)
    y = (ga_ref[...]
         + jnp.dot(gd, w1d[...], preferred_element_type=jnp.float32)
         + jnp.dot(ga, w1a[...], preferred_element_type=jnp.float32)
         + b1[...])
    y_ref[...] = y

    @pl.when(i == 0)
    def _():
        st_ref[...] = jnp.zeros_like(st_ref)

    st_ref[0:1, :] += jnp.sum(y, axis=0, keepdims=True)
    st_ref[1:2, :] += jnp.sum(y * y, axis=0, keepdims=True)


def _edge_a(gath_p, gath_np, gath_vv, dist2d, w1d, w1a, b1):
    return pl.pallas_call(
        _edge_a_body,
        grid=(E // BR,),
        in_specs=[
            pl.BlockSpec((BR, H), lambda i: (i, 0)),
            pl.BlockSpec((BR, 16), lambda i: (i, 0)),
            pl.BlockSpec((BR, 16), lambda i: (i, 0)),
            pl.BlockSpec((BR, 1), lambda i: (i, 0)),
            pl.BlockSpec((8, H), lambda i: (0, 0)),
            pl.BlockSpec((8, H), lambda i: (0, 0)),
            pl.BlockSpec((1, H), lambda i: (0, 0)),
        ],
        out_specs=[
            pl.BlockSpec((BR, H), lambda i: (i, 0)),
            pl.BlockSpec((8, H), lambda i: (0, 0)),
        ],
        out_shape=(
            jax.ShapeDtypeStruct((E, H), jnp.float32),
            jax.ShapeDtypeStruct((8, H), jnp.float32),
        ),
    )(gath_p, gath_np, gath_vv, dist2d, w1d, w1a, b1)


def _edge_c_body(y_ref, mu1, is1, w2f, b2f, w2c, b2c, u_ref):
    h = _silu((y_ref[...] - mu1[...]) * is1[...])
    zf = jnp.dot(h, w2f[...], preferred_element_type=jnp.float32) + b2f[...]
    zc = jnp.dot(h, w2c[...], preferred_element_type=jnp.float32) + b2c[...]
    gate = jax.nn.sigmoid(zf)
    sp = jnp.maximum(zc, 0.0) + jnp.log1p(jnp.exp(-jnp.abs(zc)))
    u_ref[...] = gate * sp


def _edge_c(y, mu1, is1, w2f, b2f, w2c, b2c):
    return pl.pallas_call(
        _edge_c_body,
        grid=(E // BR,),
        in_specs=[
            pl.BlockSpec((BR, H), lambda i: (i, 0)),
            pl.BlockSpec((1, H), lambda i: (0, 0)),
            pl.BlockSpec((1, H), lambda i: (0, 0)),
            pl.BlockSpec((H, H), lambda i: (0, 0)),
            pl.BlockSpec((1, H), lambda i: (0, 0)),
            pl.BlockSpec((H, H), lambda i: (0, 0)),
            pl.BlockSpec((1, H), lambda i: (0, 0)),
        ],
        out_specs=pl.BlockSpec((BR, H), lambda i: (i, 0)),
        out_shape=jax.ShapeDtypeStruct((E, H), jnp.float32),
    )(y, mu1, is1, w2f, b2f, w2c, b2c)


def _scatter_sc(u, vids):
    mesh = plsc.VectorSubcoreMesh(core_axis_name="c", subcore_axis_name="s")

    @functools.partial(
        pl.kernel,
        out_type=jax.ShapeDtypeStruct((NC, VH, H), jnp.float32),
        mesh=mesh,
        scratch_types=[
            pltpu.VMEM((CKS, H), jnp.float32),
            pltpu.VMEM((CKS,), jnp.int32),
            pltpu.VMEM_SHARED((ACC_ROWS, H), jnp.float32),
        ],
        compiler_params=pltpu.CompilerParams(use_tc_tiling_on_sc=False),
    )
    def k(u_hbm, vids_hbm, out_hbm, val_v, idx_v, acc_sh):
        cid = lax.axis_index("c")
        sid = lax.axis_index("s")
        r0 = sid * TROWS
        vbase = cid * VH

        def zbody(r, carry):
            z16 = jnp.zeros((16,), jnp.float32)
            val_v[r, 0:16] = z16
            val_v[r, 16:32] = z16
            return carry

        lax.fori_loop(0, CKS, zbody, 0)
        pltpu.sync_copy(val_v, acc_sh.at[pl.ds(r0, CKS)])
        pltpu.sync_copy(val_v.at[pl.ds(0, TROWS - CKS)],
                        acc_sh.at[pl.ds(r0 + CKS, TROWS - CKS)])

        @pl.when(sid == 0)
        def _():
            pltpu.sync_copy(val_v.at[pl.ds(0, 16)], acc_sh.at[pl.ds(VH, 16)])

        plsc.subcore_barrier()

        for i in range(NCH_S):
            base = sid * ESUB + i * CKS
            pltpu.sync_copy(u_hbm.at[pl.ds(base, CKS)], val_v)
            pltpu.sync_copy(vids_hbm.at[pl.ds(base, CKS)], idx_v)

            def xbody(j, carry):
                v = idx_v[pl.ds(j * 16, 16)] - vbase
                ok = (v >= 0) & (v < VH)
                idx_v[pl.ds(j * 16, 16)] = jnp.where(ok, v, VH)
                return carry

            lax.fori_loop(0, CKS // 16, xbody, 0)
            pltpu.sync_copy(val_v, acc_sh.at[idx_v], add=True)
        plsc.subcore_barrier()

        pltpu.sync_copy(acc_sh.at[pl.ds(r0, CKS)], val_v)
        pltpu.sync_copy(val_v, out_hbm.at[cid, pl.ds(r0, CKS)])
        rem = TROWS - CKS
        pltpu.sync_copy(acc_sh.at[pl.ds(r0 + CKS, rem)], val_v.at[pl.ds(0, rem)])
        pltpu.sync_copy(val_v.at[pl.ds(0, rem)], out_hbm.at[cid, pl.ds(r0 + CKS, rem)])

    return k(u, vids)


def _fin_a_body(a_ref, hg_ref, wf1a, wf1b, bf1, y_ref, st_ref):
    i = pl.program_id(0)
    y = (jnp.dot(a_ref[...], wf1a[...], preferred_element_type=jnp.float32)
         + jnp.dot(hg_ref[...], wf1b[...], preferred_element_type=jnp.float32)
         + bf1[...])
    y_ref[...] = y

    @pl.when(i == 0)
    def _():
        st_ref[...] = jnp.zeros_like(st_ref)

    st_ref[0:1, :] += jnp.sum(y, axis=0, keepdims=True)
    st_ref[1:2, :] += jnp.sum(y * y, axis=0, keepdims=True)


def _fin_a(a, hg, wf1a, wf1b, bf1):
    return pl.pallas_call(
        _fin_a_body,
        grid=(N_VERTS // BR,),
        in_specs=[
            pl.BlockSpec((BR, H), lambda i: (i, 0)),
            pl.BlockSpec((BR, H), lambda i: (i, 0)),
            pl.BlockSpec((H, H), lambda i: (0, 0)),
            pl.BlockSpec((H, H), lambda i: (0, 0)),
            pl.BlockSpec((1, H), lambda i: (0, 0)),
        ],
        out_specs=[
            pl.BlockSpec((BR, H), lambda i: (i, 0)),
            pl.BlockSpec((8, H), lambda i: (0, 0)),
        ],
        out_shape=(
            jax.ShapeDtypeStruct((N_VERTS, H), jnp.float32),
            jax.ShapeDtypeStruct((8, H), jnp.float32),
        ),
    )(a, hg, wf1a, wf1b, bf1)


def kernel(graph_x, node_pos, surface_x, verts, vnormals, vert_nbr_dist,
           nbr_vids, vert_nbr_ind, W_chem1, b_chem1, W_chem2, b_chem2,
           W_surf1, b_surf1, W_surf2, b_surf2, W_geom1, b_geom1, W_geom2,
           b_geom2, W_feat1, b_feat1, W_feat2, b_feat2):
    f32 = jnp.float32
    ind = vert_nbr_ind.astype(jnp.int32)
    vids = nbr_vids.astype(jnp.int32)

    wp = W_surf1[0:128]
    w1d = W_surf1[128:136]
    w1a = W_surf1[136:144]
    b1 = b_surf1.reshape(1, H)
    bc1 = b_chem1.reshape(1, H)
    bc2 = b_chem2.reshape(1, H)
    bg1 = b_geom1.reshape(1, H)
    bg2 = b_geom2.reshape(1, H)
    b2 = b_surf2.reshape(1, 2 * H)
    w2f = W_surf2[:, 0:H]
    w2c = W_surf2[:, H:2 * H]
    b2f = b_surf2[0:H].reshape(1, H)
    b2c = b_surf2[H:2 * H].reshape(1, H)
    wf1a = W_feat1[0:H]
    wf1b = W_feat1[H:2 * H]
    bf1 = b_feat1.reshape(1, H)
    bf2 = b_feat2.reshape(1, H)

    zn = jnp.zeros((N_NODES, 13), f32)
    zv = jnp.zeros((N_VERTS, 5), f32)
    tnp = jnp.concatenate([node_pos, zn], axis=1)
    tvv = jnp.concatenate([verts, zv, vnormals, zv], axis=1)

    chem_out, tp = _chem_stage(graph_x, W_chem1, bc1, W_chem2, bc2, wp)

    yg, stg1 = _lin1_stage(surface_x, W_geom1, bg1)
    mug1, isg1 = _moments(stg1, N_VERTS)
    stg2 = _mid_stage(yg, mug1, isg1, W_geom2, bg2)
    mug2, isg2 = _moments(stg2, N_VERTS)
    wg2s, bg2s = _fold_bn2(W_geom2, bg2, mug2, isg2)
    hg = _apply_stage(yg, mug1, isg1, wg2s, bg2s)

    gath_p, gath_np, gath_vv = _gather_sc(tp, tnp, tvv, ind, vids)

    dist2d = vert_nbr_dist.reshape(E, 1)
    y, st1 = _edge_a(gath_p, gath_np, gath_vv, dist2d, w1d, w1a, b1)
    mu1, is1 = _moments(st1, E)

    st2 = _mid_stage(y, mu1, is1, W_surf2, b2)
    mu2, is2 = _moments(st2, E)
    w2f_s, b2f_s = _fold_bn2(w2f, b2f, mu2[:, 0:H], is2[:, 0:H])
    w2c_s, b2c_s = _fold_bn2(w2c, b2c, mu2[:, H:2 * H], is2[:, H:2 * H])

    u = _edge_c(y, mu1, is1, w2f_s, b2f_s, w2c_s, b2c_s)

    agg2 = _scatter_sc(u, vids)
    agg = jnp.concatenate([agg2[0], agg2[1][0:N_VERTS - VH]], axis=0)

    yf, stf1 = _fin_a(agg, hg, wf1a, wf1b, bf1)
    muf1, isf1 = _moments(stf1, N_VERTS)
    stf2 = _mid_stage(yf, muf1, isf1, W_feat2, bf2)
    muf2, isf2 = _moments(stf2, N_VERTS)
    wf2s, bf2s = _fold_bn2(W_feat2, bf2, muf2, isf2)
    h_geom = _apply_stage(yf, muf1, isf1, wf2s, bf2s)

    return (h_geom, chem_out)


# speedup vs baseline: 2.7086x; 1.0314x over previous
import functools

import jax
import jax.numpy as jnp
from jax import lax
from jax.experimental import pallas as pl
from jax.experimental.pallas import tpu as pltpu
from jax.experimental.pallas import tpu_sc as plsc

H = 32
N_NODES = 10000
N_VERTS = 50000
E = 320000

NC = 2
NS = 16
NW = NC * NS
EW = E // NW
CKG = 400
NCHG = EW // CKG

VH = 25088
ACC_ROWS = VH + 16
TROWS = VH // NS
ESUB = E // NS
CKS = 800
NCH_S = ESUB // CKS

BR = 5000
BE = 8000


def _rows_block(n):
    return BE if n % BE == 0 else BR


def _bn(y):
    m = jnp.mean(y, axis=0, keepdims=True)
    v = jnp.mean((y - m) * (y - m), axis=0, keepdims=True)
    return (y - m) * lax.rsqrt(v + 1e-5)


def _gdf(x, start, stop):
    step = (stop - start) / 7.0
    c = start + step * lax.broadcasted_iota(jnp.int32, (1, 8), 1).astype(jnp.float32)
    d = x - c
    return jnp.exp(-(d * d) / (step * step))


def _silu(x):
    return x * jax.nn.sigmoid(x)


def _chem_body(gx_ref, wc1, bc1, wc2, bc2, wp, chem_ref, tp_ref):
    gx = gx_ref[...]
    h = _silu(_bn(jnp.dot(gx, wc1[...], preferred_element_type=jnp.float32) + bc1[...]))
    chem_ref[...] = _bn(jnp.dot(h, wc2[...], preferred_element_type=jnp.float32) + bc2[...])
    tp_ref[...] = jnp.dot(gx, wp[...], preferred_element_type=jnp.float32)


def _chem_stage(graph_x, wc1, bc1, wc2, bc2, wp):
    return pl.pallas_call(
        _chem_body,
        out_shape=(
            jax.ShapeDtypeStruct((N_NODES, H), jnp.float32),
            jax.ShapeDtypeStruct((N_NODES, H), jnp.float32),
        ),
    )(graph_x, wc1, bc1, wc2, bc2, wp)


def _lin1_body(x_ref, w1, b1, y_ref, st_ref):
    i = pl.program_id(0)
    y = jnp.dot(x_ref[...], w1[...], preferred_element_type=jnp.float32) + b1[...]
    y_ref[...] = y

    @pl.when(i == 0)
    def _():
        st_ref[...] = jnp.zeros_like(st_ref)

    st_ref[0:1, :] += jnp.sum(y, axis=0, keepdims=True)
    st_ref[1:2, :] += jnp.sum(y * y, axis=0, keepdims=True)


def _lin1_stage(x, w1, b1):
    n, fi = x.shape
    return pl.pallas_call(
        _lin1_body,
        grid=(n // BR,),
        in_specs=[
            pl.BlockSpec((BR, fi), lambda i: (i, 0)),
            pl.BlockSpec((fi, H), lambda i: (0, 0)),
            pl.BlockSpec((1, H), lambda i: (0, 0)),
        ],
        out_specs=[
            pl.BlockSpec((BR, H), lambda i: (i, 0)),
            pl.BlockSpec((8, H), lambda i: (0, 0)),
        ],
        out_shape=(
            jax.ShapeDtypeStruct((n, H), jnp.float32),
            jax.ShapeDtypeStruct((8, H), jnp.float32),
        ),
    )(x, w1, b1)


def _mid_body(y_ref, mu1, is1, w2, b2, st_ref):
    i = pl.program_id(0)
    h = _silu((y_ref[...] - mu1[...]) * is1[...])
    z = jnp.dot(h, w2[...], preferred_element_type=jnp.float32) + b2[...]

    @pl.when(i == 0)
    def _():
        st_ref[...] = jnp.zeros_like(st_ref)

    st_ref[0:1, :] += jnp.sum(z, axis=0, keepdims=True)
    st_ref[1:2, :] += jnp.sum(z * z, axis=0, keepdims=True)


def _mid_stage(y, mu1, is1, w2, b2):
    n, _ = y.shape
    wo = w2.shape[1]
    br = _rows_block(n)
    return pl.pallas_call(
        _mid_body,
        grid=(n // br,),
        in_specs=[
            pl.BlockSpec((br, H), lambda i: (i, 0)),
            pl.BlockSpec((1, H), lambda i: (0, 0)),
            pl.BlockSpec((1, H), lambda i: (0, 0)),
            pl.BlockSpec((H, wo), lambda i: (0, 0)),
            pl.BlockSpec((1, wo), lambda i: (0, 0)),
        ],
        out_specs=pl.BlockSpec((8, wo), lambda i: (0, 0)),
        out_shape=jax.ShapeDtypeStruct((8, wo), jnp.float32),
    )(y, mu1, is1, w2, b2)


def _apply_body(y_ref, mu1, is1, w2s, b2s, out_ref):
    h = _silu((y_ref[...] - mu1[...]) * is1[...])
    out_ref[...] = jnp.dot(h, w2s[...], preferred_element_type=jnp.float32) + b2s[...]


def _apply_stage(y, mu1, is1, w2s, b2s):
    n, _ = y.shape
    br = _rows_block(n)
    return pl.pallas_call(
        _apply_body,
        grid=(n // br,),
        in_specs=[
            pl.BlockSpec((br, H), lambda i: (i, 0)),
            pl.BlockSpec((1, H), lambda i: (0, 0)),
            pl.BlockSpec((1, H), lambda i: (0, 0)),
            pl.BlockSpec((H, H), lambda i: (0, 0)),
            pl.BlockSpec((1, H), lambda i: (0, 0)),
        ],
        out_specs=pl.BlockSpec((br, H), lambda i: (i, 0)),
        out_shape=jax.ShapeDtypeStruct((n, H), jnp.float32),
    )(y, mu1, is1, w2s, b2s)


def _moments(st, n):
    mu = st[0] / n
    var = st[1] / n - mu * mu
    return mu.reshape(1, -1), lax.rsqrt(var + 1e-5).reshape(1, -1)


def _fold_bn2(w2, b2, mu2, is2):
    return w2 * is2, (b2 - mu2) * is2


def _gather_sc(tp, tnp, tvv, ind, vids):
    mesh = plsc.VectorSubcoreMesh(core_axis_name="c", subcore_axis_name="s")

    @functools.partial(
        pl.kernel,
        out_type=(
            jax.ShapeDtypeStruct((E, H), jnp.float32),
            jax.ShapeDtypeStruct((E, 16), jnp.float32),
            jax.ShapeDtypeStruct((E, 16), jnp.float32),
        ),
        mesh=mesh,
        scratch_types=(
            [
                pltpu.VMEM((2, CKG), jnp.int32),
                pltpu.VMEM((2, CKG), jnp.int32),
                pltpu.VMEM((2, CKG, H), jnp.float32),
                pltpu.VMEM((2, CKG, 16), jnp.float32),
                pltpu.VMEM((2, CKG, 16), jnp.float32),
            ]
            + [pltpu.SemaphoreType.DMA] * 12
        ),
        compiler_params=pltpu.CompilerParams(use_tc_tiling_on_sc=False),
    )
    def k(tp_hbm, tnp_hbm, tvv_hbm, ind_hbm, vids_hbm, oa_hbm, ob_hbm, oc_hbm,
          idx1_v, idx2_v, ra_v, rb_v, rc_v, *sems):
        wid = lax.axis_index("s") * NC + lax.axis_index("c")
        gsem = [sems[0:2], sems[2:4], sems[4:6]]
        wsem = [sems[6:8], sems[8:10], sems[10:12]]
        gd = {}
        wd = {}

        def start_chunk(i):
            b = i & 1
            base = wid * EW + i * CKG
            pltpu.sync_copy(ind_hbm.at[pl.ds(base, CKG)], idx1_v.at[b])
            pltpu.sync_copy(vids_hbm.at[pl.ds(base, CKG)], idx2_v.at[b])
            gd[i] = (
                pltpu.async_copy(tp_hbm.at[idx1_v.at[b]], ra_v.at[b], gsem[0][b]),
                pltpu.async_copy(tnp_hbm.at[idx1_v.at[b]], rb_v.at[b], gsem[1][b]),
                pltpu.async_copy(tvv_hbm.at[idx2_v.at[b]], rc_v.at[b], gsem[2][b]),
            )

        start_chunk(0)
        for i in range(NCHG):
            b = i & 1
            base = wid * EW + i * CKG
            for d in gd.pop(i):
                d.wait()
            wd[i] = (
                pltpu.async_copy(ra_v.at[b], oa_hbm.at[pl.ds(base, CKG)], wsem[0][b]),
                pltpu.async_copy(rb_v.at[b], ob_hbm.at[pl.ds(base, CKG)], wsem[1][b]),
                pltpu.async_copy(rc_v.at[b], oc_hbm.at[pl.ds(base, CKG)], wsem[2][b]),
            )
            if i + 1 < NCHG:
                if i - 1 >= 0:
                    for d in wd.pop(i - 1):
                        d.wait()
                start_chunk(i + 1)
        for d in wd.pop(NCHG - 1):
            d.wait()

    return k(tp, tnp, tvv, ind, vids)


def _edge_a_body(ga_ref, gnp_ref, gvv_ref, d_ref, w1da, b1, y_ref, st_ref):
    i = pl.program_id(0)
    p8 = gnp_ref[:, 0:8]
    v8 = gvv_ref[:, 0:8]
    n8 = gvv_ref[:, 8:16]
    dlt = p8 - v8
    ones8 = jnp.full((8, 1), 1.0, jnp.float32)
    nrm2 = jnp.dot(dlt * dlt, ones8, preferred_element_type=jnp.float32)
    dot = jnp.dot(dlt * n8, ones8, preferred_element_type=jnp.float32)
    ang = dot * lax.rsqrt(nrm2)
    gda = jnp.concatenate([_gdf(d_ref[...], 0.0, 8.0), _gdf(ang, -1.0, 1.0)],
                          axis=1)
    y = (ga_ref[...]
         + jnp.dot(gda, w1da[...], preferred_element_type=jnp.float32)
         + b1[...])
    y_ref[...] = y

    @pl.when(i == 0)
    def _():
        st_ref[...] = jnp.zeros_like(st_ref)

    st_ref[0:1, :] += jnp.sum(y, axis=0, keepdims=True)
    st_ref[1:2, :] += jnp.sum(y * y, axis=0, keepdims=True)


def _edge_a(gath_p, gath_np, gath_vv, dist2d, w1da, b1):
    return pl.pallas_call(
        _edge_a_body,
        grid=(E // BE,),
        in_specs=[
            pl.BlockSpec((BE, H), lambda i: (i, 0)),
            pl.BlockSpec((BE, 16), lambda i: (i, 0)),
            pl.BlockSpec((BE, 16), lambda i: (i, 0)),
            pl.BlockSpec((BE, 1), lambda i: (i, 0)),
            pl.BlockSpec((16, H), lambda i: (0, 0)),
            pl.BlockSpec((1, H), lambda i: (0, 0)),
        ],
        out_specs=[
            pl.BlockSpec((BE, H), lambda i: (i, 0)),
            pl.BlockSpec((8, H), lambda i: (0, 0)),
        ],
        out_shape=(
            jax.ShapeDtypeStruct((E, H), jnp.float32),
            jax.ShapeDtypeStruct((8, H), jnp.float32),
        ),
    )(gath_p, gath_np, gath_vv, dist2d, w1da, b1)


def _edge_c_body(y_ref, mu1, is1, w2f, b2f, w2c, b2c, u_ref):
    h = _silu((y_ref[...] - mu1[...]) * is1[...])
    zf = jnp.dot(h, w2f[...], preferred_element_type=jnp.float32) + b2f[...]
    zc = jnp.dot(h, w2c[...], preferred_element_type=jnp.float32) + b2c[...]
    gate = jax.nn.sigmoid(zf)
    sp = jnp.maximum(zc, 0.0) + jnp.log1p(jnp.exp(-jnp.abs(zc)))
    u_ref[...] = gate * sp


def _edge_c(y, mu1, is1, w2f, b2f, w2c, b2c):
    return pl.pallas_call(
        _edge_c_body,
        grid=(E // BE,),
        in_specs=[
            pl.BlockSpec((BE, H), lambda i: (i, 0)),
            pl.BlockSpec((1, H), lambda i: (0, 0)),
            pl.BlockSpec((1, H), lambda i: (0, 0)),
            pl.BlockSpec((H, H), lambda i: (0, 0)),
            pl.BlockSpec((1, H), lambda i: (0, 0)),
            pl.BlockSpec((H, H), lambda i: (0, 0)),
            pl.BlockSpec((1, H), lambda i: (0, 0)),
        ],
        out_specs=pl.BlockSpec((BE, H), lambda i: (i, 0)),
        out_shape=jax.ShapeDtypeStruct((E, H), jnp.float32),
    )(y, mu1, is1, w2f, b2f, w2c, b2c)


def _scatter_sc(u, vids):
    mesh = plsc.VectorSubcoreMesh(core_axis_name="c", subcore_axis_name="s")

    @functools.partial(
        pl.kernel,
        out_type=jax.ShapeDtypeStruct((NC, VH, H), jnp.float32),
        mesh=mesh,
        scratch_types=[
            pltpu.VMEM((CKS, H), jnp.float32),
            pltpu.VMEM((CKS,), jnp.int32),
            pltpu.VMEM_SHARED((ACC_ROWS, H), jnp.float32),
        ],
        compiler_params=pltpu.CompilerParams(use_tc_tiling_on_sc=False),
    )
    def k(u_hbm, vids_hbm, out_hbm, val_v, idx_v, acc_sh):
        cid = lax.axis_index("c")
        sid = lax.axis_index("s")
        r0 = sid * TROWS
        vbase = cid * VH

        def zbody(r, carry):
            z16 = jnp.zeros((16,), jnp.float32)
            val_v[r, 0:16] = z16
            val_v[r, 16:32] = z16
            return carry

        lax.fori_loop(0, CKS, zbody, 0)
        pltpu.sync_copy(val_v, acc_sh.at[pl.ds(r0, CKS)])
        pltpu.sync_copy(val_v.at[pl.ds(0, TROWS - CKS)],
                        acc_sh.at[pl.ds(r0 + CKS, TROWS - CKS)])

        @pl.when(sid == 0)
        def _():
            pltpu.sync_copy(val_v.at[pl.ds(0, 16)], acc_sh.at[pl.ds(VH, 16)])

        plsc.subcore_barrier()

        for i in range(NCH_S):
            base = sid * ESUB + i * CKS
            pltpu.sync_copy(u_hbm.at[pl.ds(base, CKS)], val_v)
            pltpu.sync_copy(vids_hbm.at[pl.ds(base, CKS)], idx_v)

            def xbody(j, carry):
                v = idx_v[pl.ds(j * 16, 16)] - vbase
                ok = (v >= 0) & (v < VH)
                idx_v[pl.ds(j * 16, 16)] = jnp.where(ok, v, VH)
                return carry

            lax.fori_loop(0, CKS // 16, xbody, 0)
            pltpu.sync_copy(val_v, acc_sh.at[idx_v], add=True)
        plsc.subcore_barrier()

        pltpu.sync_copy(acc_sh.at[pl.ds(r0, CKS)], val_v)
        pltpu.sync_copy(val_v, out_hbm.at[cid, pl.ds(r0, CKS)])
        rem = TROWS - CKS
        pltpu.sync_copy(acc_sh.at[pl.ds(r0 + CKS, rem)], val_v.at[pl.ds(0, rem)])
        pltpu.sync_copy(val_v.at[pl.ds(0, rem)], out_hbm.at[cid, pl.ds(r0 + CKS, rem)])

    return k(u, vids)


def _fin_a_body(a_ref, hg_ref, wf1a, wf1b, bf1, y_ref, st_ref):
    i = pl.program_id(0)
    y = (jnp.dot(a_ref[...], wf1a[...], preferred_element_type=jnp.float32)
         + jnp.dot(hg_ref[...], wf1b[...], preferred_element_type=jnp.float32)
         + bf1[...])
    y_ref[...] = y

    @pl.when(i == 0)
    def _():
        st_ref[...] = jnp.zeros_like(st_ref)

    st_ref[0:1, :] += jnp.sum(y, axis=0, keepdims=True)
    st_ref[1:2, :] += jnp.sum(y * y, axis=0, keepdims=True)


def _fin_a(a, hg, wf1a, wf1b, bf1):
    return pl.pallas_call(
        _fin_a_body,
        grid=(N_VERTS // BR,),
        in_specs=[
            pl.BlockSpec((BR, H), lambda i: (i, 0)),
            pl.BlockSpec((BR, H), lambda i: (i, 0)),
            pl.BlockSpec((H, H), lambda i: (0, 0)),
            pl.BlockSpec((H, H), lambda i: (0, 0)),
            pl.BlockSpec((1, H), lambda i: (0, 0)),
        ],
        out_specs=[
            pl.BlockSpec((BR, H), lambda i: (i, 0)),
            pl.BlockSpec((8, H), lambda i: (0, 0)),
        ],
        out_shape=(
            jax.ShapeDtypeStruct((N_VERTS, H), jnp.float32),
            jax.ShapeDtypeStruct((8, H), jnp.float32),
        ),
    )(a, hg, wf1a, wf1b, bf1)


def kernel(graph_x, node_pos, surface_x, verts, vnormals, vert_nbr_dist,
           nbr_vids, vert_nbr_ind, W_chem1, b_chem1, W_chem2, b_chem2,
           W_surf1, b_surf1, W_surf2, b_surf2, W_geom1, b_geom1, W_geom2,
           b_geom2, W_feat1, b_feat1, W_feat2, b_feat2):
    f32 = jnp.float32
    ind = vert_nbr_ind.astype(jnp.int32)
    vids = nbr_vids.astype(jnp.int32)

    wp = W_surf1[0:128]
    w1da = W_surf1[128:144]
    b1 = b_surf1.reshape(1, H)
    bc1 = b_chem1.reshape(1, H)
    bc2 = b_chem2.reshape(1, H)
    bg1 = b_geom1.reshape(1, H)
    bg2 = b_geom2.reshape(1, H)
    b2 = b_surf2.reshape(1, 2 * H)
    w2f = W_surf2[:, 0:H]
    w2c = W_surf2[:, H:2 * H]
    b2f = b_surf2[0:H].reshape(1, H)
    b2c = b_surf2[H:2 * H].reshape(1, H)
    wf1a = W_feat1[0:H]
    wf1b = W_feat1[H:2 * H]
    bf1 = b_feat1.reshape(1, H)
    bf2 = b_feat2.reshape(1, H)

    zn = jnp.zeros((N_NODES, 13), f32)
    zv = jnp.zeros((N_VERTS, 5), f32)
    tnp = jnp.concatenate([node_pos, zn], axis=1)
    tvv = jnp.concatenate([verts, zv, vnormals, zv], axis=1)

    chem_out, tp = _chem_stage(graph_x, W_chem1, bc1, W_chem2, bc2, wp)

    yg, stg1 = _lin1_stage(surface_x, W_geom1, bg1)
    mug1, isg1 = _moments(stg1, N_VERTS)
    stg2 = _mid_stage(yg, mug1, isg1, W_geom2, bg2)
    mug2, isg2 = _moments(stg2, N_VERTS)
    wg2s, bg2s = _fold_bn2(W_geom2, bg2, mug2, isg2)
    hg = _apply_stage(yg, mug1, isg1, wg2s, bg2s)

    gath_p, gath_np, gath_vv = _gather_sc(tp, tnp, tvv, ind, vids)

    dist2d = vert_nbr_dist.reshape(E, 1)
    y, st1 = _edge_a(gath_p, gath_np, gath_vv, dist2d, w1da, b1)
    mu1, is1 = _moments(st1, E)

    st2 = _mid_stage(y, mu1, is1, W_surf2, b2)
    mu2, is2 = _moments(st2, E)
    w2f_s, b2f_s = _fold_bn2(w2f, b2f, mu2[:, 0:H], is2[:, 0:H])
    w2c_s, b2c_s = _fold_bn2(w2c, b2c, mu2[:, H:2 * H], is2[:, H:2 * H])

    u = _edge_c(y, mu1, is1, w2f_s, b2f_s, w2c_s, b2c_s)

    agg2 = _scatter_sc(u, vids)
    agg = jnp.concatenate([agg2[0], agg2[1][0:N_VERTS - VH]], axis=0)

    yf, stf1 = _fin_a(agg, hg, wf1a, wf1b, bf1)
    muf1, isf1 = _moments(stf1, N_VERTS)
    stf2 = _mid_stage(yf, muf1, isf1, W_feat2, bf2)
    muf2, isf2 = _moments(stf2, N_VERTS)
    wf2s, bf2s = _fold_bn2(W_feat2, bf2, muf2, isf2)
    h_geom = _apply_stage(yf, muf1, isf1, wf2s, bf2s)

    return (h_geom, chem_out)


# speedup vs baseline: 2.7307x; 1.0081x over previous
import functools

import jax
import jax.numpy as jnp
from jax import lax
from jax.experimental import pallas as pl
from jax.experimental.pallas import tpu as pltpu
from jax.experimental.pallas import tpu_sc as plsc

H = 32
N_NODES = 10000
N_VERTS = 50000
E = 320000

NC = 2
NS = 16
NW = NC * NS
EW = E // NW
CKG = 400
NCHG = EW // CKG

VH = 25088
ACC_ROWS = VH + 16
TROWS = VH // NS
ESUB = E // NS
CKS = 800
NCH_S = ESUB // CKS

BR = 5000
BE = 8000


def _rows_block(n):
    return BE if n % BE == 0 else BR


def _bn(y):
    m = jnp.mean(y, axis=0, keepdims=True)
    v = jnp.mean((y - m) * (y - m), axis=0, keepdims=True)
    return (y - m) * lax.rsqrt(v + 1e-5)


def _gdf(x, start, stop):
    step = (stop - start) / 7.0
    c = start + step * lax.broadcasted_iota(jnp.int32, (1, 8), 1).astype(jnp.float32)
    d = x - c
    return jnp.exp(-(d * d) / (step * step))


def _silu(x):
    return x * jax.nn.sigmoid(x)


def _chem_body(gx_ref, np_ref, wc1, bc1, wc2, bc2, wp, chem_ref, tp_ref):
    gx = gx_ref[...]
    h = _silu(_bn(jnp.dot(gx, wc1[...], preferred_element_type=jnp.float32) + bc1[...]))
    chem_ref[...] = _bn(jnp.dot(h, wc2[...], preferred_element_type=jnp.float32) + bc2[...])
    proj = jnp.dot(gx, wp[...], preferred_element_type=jnp.float32)
    pad = jnp.zeros((N_NODES, 13), jnp.float32)
    tp_ref[...] = jnp.concatenate([proj, np_ref[...], pad], axis=1)


def _chem_stage(graph_x, node_pos, wc1, bc1, wc2, bc2, wp):
    return pl.pallas_call(
        _chem_body,
        out_shape=(
            jax.ShapeDtypeStruct((N_NODES, H), jnp.float32),
            jax.ShapeDtypeStruct((N_NODES, 48), jnp.float32),
        ),
    )(graph_x, node_pos, wc1, bc1, wc2, bc2, wp)


def _lin1_body(x_ref, w1, b1, y_ref, st_ref):
    i = pl.program_id(0)
    y = jnp.dot(x_ref[...], w1[...], preferred_element_type=jnp.float32) + b1[...]
    y_ref[...] = y

    @pl.when(i == 0)
    def _():
        st_ref[...] = jnp.zeros_like(st_ref)

    st_ref[0:1, :] += jnp.sum(y, axis=0, keepdims=True)
    st_ref[1:2, :] += jnp.sum(y * y, axis=0, keepdims=True)


def _lin1_stage(x, w1, b1):
    n, fi = x.shape
    return pl.pallas_call(
        _lin1_body,
        grid=(n // BR,),
        in_specs=[
            pl.BlockSpec((BR, fi), lambda i: (i, 0)),
            pl.BlockSpec((fi, H), lambda i: (0, 0)),
            pl.BlockSpec((1, H), lambda i: (0, 0)),
        ],
        out_specs=[
            pl.BlockSpec((BR, H), lambda i: (i, 0)),
            pl.BlockSpec((8, H), lambda i: (0, 0)),
        ],
        out_shape=(
            jax.ShapeDtypeStruct((n, H), jnp.float32),
            jax.ShapeDtypeStruct((8, H), jnp.float32),
        ),
    )(x, w1, b1)


def _st_moments(st_ref, n):
    mu = st_ref[0:1, :] * (1.0 / n)
    var = st_ref[1:2, :] * (1.0 / n) - mu * mu
    return mu, lax.rsqrt(var + 1e-5)


def _mid_body(n, y_ref, st1, w2, b2, st_ref):
    i = pl.program_id(0)
    mu1, is1 = _st_moments(st1, n)
    h = _silu((y_ref[...] - mu1) * is1)
    z = jnp.dot(h, w2[...], preferred_element_type=jnp.float32) + b2[...]

    @pl.when(i == 0)
    def _():
        st_ref[...] = jnp.zeros_like(st_ref)

    st_ref[0:1, :] += jnp.sum(z, axis=0, keepdims=True)
    st_ref[1:2, :] += jnp.sum(z * z, axis=0, keepdims=True)


def _mid_stage(y, st1, w2, b2):
    n, _ = y.shape
    wo = w2.shape[1]
    br = _rows_block(n)
    return pl.pallas_call(
        functools.partial(_mid_body, n),
        grid=(n // br,),
        in_specs=[
            pl.BlockSpec((br, H), lambda i: (i, 0)),
            pl.BlockSpec((8, H), lambda i: (0, 0)),
            pl.BlockSpec((H, wo), lambda i: (0, 0)),
            pl.BlockSpec((1, wo), lambda i: (0, 0)),
        ],
        out_specs=pl.BlockSpec((8, wo), lambda i: (0, 0)),
        out_shape=jax.ShapeDtypeStruct((8, wo), jnp.float32),
    )(y, st1, w2, b2)


def _apply_body(n, y_ref, st1, st2, w2, b2, out_ref):
    mu1, is1 = _st_moments(st1, n)
    mu2, is2 = _st_moments(st2, n)
    h = _silu((y_ref[...] - mu1) * is1)
    z = jnp.dot(h, w2[...] * is2, preferred_element_type=jnp.float32)
    out_ref[...] = z + (b2[...] - mu2) * is2


def _apply_stage(y, st1, st2, w2, b2):
    n, _ = y.shape
    br = _rows_block(n)
    return pl.pallas_call(
        functools.partial(_apply_body, n),
        grid=(n // br,),
        in_specs=[
            pl.BlockSpec((br, H), lambda i: (i, 0)),
            pl.BlockSpec((8, H), lambda i: (0, 0)),
            pl.BlockSpec((8, H), lambda i: (0, 0)),
            pl.BlockSpec((H, H), lambda i: (0, 0)),
            pl.BlockSpec((1, H), lambda i: (0, 0)),
        ],
        out_specs=pl.BlockSpec((br, H), lambda i: (i, 0)),
        out_shape=jax.ShapeDtypeStruct((n, H), jnp.float32),
    )(y, st1, st2, w2, b2)


def _gather_sc(tp, tvv, ind, vids):
    mesh = plsc.VectorSubcoreMesh(core_axis_name="c", subcore_axis_name="s")

    @functools.partial(
        pl.kernel,
        out_type=(
            jax.ShapeDtypeStruct((E, 48), jnp.float32),
            jax.ShapeDtypeStruct((E, 16), jnp.float32),
        ),
        mesh=mesh,
        scratch_types=(
            [
                pltpu.VMEM((2, CKG), jnp.int32),
                pltpu.VMEM((2, CKG), jnp.int32),
                pltpu.VMEM((2, CKG, 48), jnp.float32),
                pltpu.VMEM((2, CKG, 16), jnp.float32),
            ]
            + [pltpu.SemaphoreType.DMA] * 8
        ),
        compiler_params=pltpu.CompilerParams(use_tc_tiling_on_sc=False),
    )
    def k(tp_hbm, tvv_hbm, ind_hbm, vids_hbm, oa_hbm, oc_hbm,
          idx1_v, idx2_v, ra_v, rc_v, *sems):
        wid = lax.axis_index("s") * NC + lax.axis_index("c")
        gsem = [sems[0:2], sems[2:4]]
        wsem = [sems[4:6], sems[6:8]]
        gd = {}
        wd = {}

        def start_chunk(i):
            b = i & 1
            base = wid * EW + i * CKG
            pltpu.sync_copy(ind_hbm.at[pl.ds(base, CKG)], idx1_v.at[b])
            pltpu.sync_copy(vids_hbm.at[pl.ds(base, CKG)], idx2_v.at[b])
            gd[i] = (
                pltpu.async_copy(tp_hbm.at[idx1_v.at[b]], ra_v.at[b], gsem[0][b]),
                pltpu.async_copy(tvv_hbm.at[idx2_v.at[b]], rc_v.at[b], gsem[1][b]),
            )

        start_chunk(0)
        for i in range(NCHG):
            b = i & 1
            base = wid * EW + i * CKG
            for d in gd.pop(i):
                d.wait()
            wd[i] = (
                pltpu.async_copy(ra_v.at[b], oa_hbm.at[pl.ds(base, CKG)], wsem[0][b]),
                pltpu.async_copy(rc_v.at[b], oc_hbm.at[pl.ds(base, CKG)], wsem[1][b]),
            )
            if i + 1 < NCHG:
                if i - 1 >= 0:
                    for d in wd.pop(i - 1):
                        d.wait()
                start_chunk(i + 1)
        for d in wd.pop(NCHG - 1):
            d.wait()

    return k(tp, tvv, ind, vids)


def _edge_a_body(ga_ref, gvv_ref, d_ref, w1da, b1, y_ref, st_ref):
    i = pl.program_id(0)
    p8 = ga_ref[:, 32:40]
    v8 = gvv_ref[:, 0:8]
    n8 = gvv_ref[:, 8:16]
    dlt = p8 - v8
    ones8 = jnp.full((8, 1), 1.0, jnp.float32)
    nrm2 = jnp.dot(dlt * dlt, ones8, preferred_element_type=jnp.float32)
    dot = jnp.dot(dlt * n8, ones8, preferred_element_type=jnp.float32)
    ang = dot * lax.rsqrt(nrm2)
    gda = jnp.concatenate([_gdf(d_ref[...], 0.0, 8.0), _gdf(ang, -1.0, 1.0)],
                          axis=1)
    y = (ga_ref[:, 0:32]
         + jnp.dot(gda, w1da[...], preferred_element_type=jnp.float32)
         + b1[...])
    y_ref[...] = y

    @pl.when(i == 0)
    def _():
        st_ref[...] = jnp.zeros_like(st_ref)

    st_ref[0:1, :] += jnp.sum(y, axis=0, keepdims=True)
    st_ref[1:2, :] += jnp.sum(y * y, axis=0, keepdims=True)


def _edge_a(gath_p, gath_vv, dist2d, w1da, b1):
    return pl.pallas_call(
        _edge_a_body,
        grid=(E // BE,),
        in_specs=[
            pl.BlockSpec((BE, 48), lambda i: (i, 0)),
            pl.BlockSpec((BE, 16), lambda i: (i, 0)),
            pl.BlockSpec((BE, 1), lambda i: (i, 0)),
            pl.BlockSpec((16, H), lambda i: (0, 0)),
            pl.BlockSpec((1, H), lambda i: (0, 0)),
        ],
        out_specs=[
            pl.BlockSpec((BE, H), lambda i: (i, 0)),
            pl.BlockSpec((8, H), lambda i: (0, 0)),
        ],
        out_shape=(
            jax.ShapeDtypeStruct((E, H), jnp.float32),
            jax.ShapeDtypeStruct((8, H), jnp.float32),
        ),
    )(gath_p, gath_vv, dist2d, w1da, b1)


def _edge_c_body(y_ref, st1, st2, w2f, b2f, w2c, b2c, u_ref):
    mu1, is1 = _st_moments(st1, E)
    mu2, is2 = _st_moments(st2, E)
    mu2f, mu2c = mu2[:, 0:H], mu2[:, H:2 * H]
    is2f, is2c = is2[:, 0:H], is2[:, H:2 * H]
    h = _silu((y_ref[...] - mu1) * is1)
    zf = (jnp.dot(h, w2f[...] * is2f, preferred_element_type=jnp.float32)
          + (b2f[...] - mu2f) * is2f)
    zc = (jnp.dot(h, w2c[...] * is2c, preferred_element_type=jnp.float32)
          + (b2c[...] - mu2c) * is2c)
    gate = jax.nn.sigmoid(zf)
    sp = jnp.maximum(zc, 0.0) + jnp.log1p(jnp.exp(-jnp.abs(zc)))
    u_ref[...] = gate * sp


def _edge_c(y, st1, st2, w2f, b2f, w2c, b2c):
    return pl.pallas_call(
        _edge_c_body,
        grid=(E // BE,),
        in_specs=[
            pl.BlockSpec((BE, H), lambda i: (i, 0)),
            pl.BlockSpec((8, H), lambda i: (0, 0)),
            pl.BlockSpec((8, 2 * H), lambda i: (0, 0)),
            pl.BlockSpec((H, H), lambda i: (0, 0)),
            pl.BlockSpec((1, H), lambda i: (0, 0)),
            pl.BlockSpec((H, H), lambda i: (0, 0)),
            pl.BlockSpec((1, H), lambda i: (0, 0)),
        ],
        out_specs=pl.BlockSpec((BE, H), lambda i: (i, 0)),
        out_shape=jax.ShapeDtypeStruct((E, H), jnp.float32),
    )(y, st1, st2, w2f, b2f, w2c, b2c)


def _scatter_sc(u, vids):
    mesh = plsc.VectorSubcoreMesh(core_axis_name="c", subcore_axis_name="s")

    @functools.partial(
        pl.kernel,
        out_type=jax.ShapeDtypeStruct((NC * VH, H), jnp.float32),
        mesh=mesh,
        scratch_types=[
            pltpu.VMEM((CKS, H), jnp.float32),
            pltpu.VMEM((CKS,), jnp.int32),
            pltpu.VMEM_SHARED((ACC_ROWS, H), jnp.float32),
        ],
        compiler_params=pltpu.CompilerParams(use_tc_tiling_on_sc=False),
    )
    def k(u_hbm, vids_hbm, out_hbm, val_v, idx_v, acc_sh):
        cid = lax.axis_index("c")
        sid = lax.axis_index("s")
        r0 = sid * TROWS
        vbase = cid * VH

        def zbody(r, carry):
            z16 = jnp.zeros((16,), jnp.float32)
            val_v[r, 0:16] = z16
            val_v[r, 16:32] = z16
            return carry

        lax.fori_loop(0, CKS, zbody, 0)
        pltpu.sync_copy(val_v, acc_sh.at[pl.ds(r0, CKS)])
        pltpu.sync_copy(val_v.at[pl.ds(0, TROWS - CKS)],
                        acc_sh.at[pl.ds(r0 + CKS, TROWS - CKS)])

        @pl.when(sid == 0)
        def _():
            pltpu.sync_copy(val_v.at[pl.ds(0, 16)], acc_sh.at[pl.ds(VH, 16)])

        plsc.subcore_barrier()

        for i in range(NCH_S):
            base = sid * ESUB + i * CKS
            pltpu.sync_copy(u_hbm.at[pl.ds(base, CKS)], val_v)
            pltpu.sync_copy(vids_hbm.at[pl.ds(base, CKS)], idx_v)

            def xbody(j, carry):
                v = idx_v[pl.ds(j * 16, 16)] - vbase
                ok = (v >= 0) & (v < VH)
                idx_v[pl.ds(j * 16, 16)] = jnp.where(ok, v, VH)
                return carry

            lax.fori_loop(0, CKS // 16, xbody, 0)
            pltpu.sync_copy(val_v, acc_sh.at[idx_v], add=True)
        plsc.subcore_barrier()

        o0 = vbase + r0
        pltpu.sync_copy(acc_sh.at[pl.ds(r0, CKS)], val_v)
        pltpu.sync_copy(val_v, out_hbm.at[pl.ds(o0, CKS)])
        rem = TROWS - CKS
        pltpu.sync_copy(acc_sh.at[pl.ds(r0 + CKS, rem)], val_v.at[pl.ds(0, rem)])
        pltpu.sync_copy(val_v.at[pl.ds(0, rem)], out_hbm.at[pl.ds(o0 + CKS, rem)])

    return k(u, vids)


def _fin_a_body(a_ref, hg_ref, wf1a, wf1b, bf1, y_ref, st_ref):
    i = pl.program_id(0)
    y = (jnp.dot(a_ref[...], wf1a[...], preferred_element_type=jnp.float32)
         + jnp.dot(hg_ref[...], wf1b[...], preferred_element_type=jnp.float32)
         + bf1[...])
    y_ref[...] = y

    @pl.when(i == 0)
    def _():
        st_ref[...] = jnp.zeros_like(st_ref)

    st_ref[0:1, :] += jnp.sum(y, axis=0, keepdims=True)
    st_ref[1:2, :] += jnp.sum(y * y, axis=0, keepdims=True)


def _fin_a(a, hg, wf1a, wf1b, bf1):
    return pl.pallas_call(
        _fin_a_body,
        grid=(N_VERTS // BR,),
        in_specs=[
            pl.BlockSpec((BR, H), lambda i: (i, 0)),
            pl.BlockSpec((BR, H), lambda i: (i, 0)),
            pl.BlockSpec((H, H), lambda i: (0, 0)),
            pl.BlockSpec((H, H), lambda i: (0, 0)),
            pl.BlockSpec((1, H), lambda i: (0, 0)),
        ],
        out_specs=[
            pl.BlockSpec((BR, H), lambda i: (i, 0)),
            pl.BlockSpec((8, H), lambda i: (0, 0)),
        ],
        out_shape=(
            jax.ShapeDtypeStruct((N_VERTS, H), jnp.float32),
            jax.ShapeDtypeStruct((8, H), jnp.float32),
        ),
    )(a, hg, wf1a, wf1b, bf1)


def kernel(graph_x, node_pos, surface_x, verts, vnormals, vert_nbr_dist,
           nbr_vids, vert_nbr_ind, W_chem1, b_chem1, W_chem2, b_chem2,
           W_surf1, b_surf1, W_surf2, b_surf2, W_geom1, b_geom1, W_geom2,
           b_geom2, W_feat1, b_feat1, W_feat2, b_feat2):
    f32 = jnp.float32
    ind = vert_nbr_ind.astype(jnp.int32)
    vids = nbr_vids.astype(jnp.int32)

    wp = W_surf1[0:128]
    w1da = W_surf1[128:144]
    b1 = b_surf1.reshape(1, H)
    bc1 = b_chem1.reshape(1, H)
    bc2 = b_chem2.reshape(1, H)
    bg1 = b_geom1.reshape(1, H)
    bg2 = b_geom2.reshape(1, H)
    b2 = b_surf2.reshape(1, 2 * H)
    w2f = W_surf2[:, 0:H]
    w2c = W_surf2[:, H:2 * H]
    b2f = b_surf2[0:H].reshape(1, H)
    b2c = b_surf2[H:2 * H].reshape(1, H)
    wf1a = W_feat1[0:H]
    wf1b = W_feat1[H:2 * H]
    bf1 = b_feat1.reshape(1, H)
    bf2 = b_feat2.reshape(1, H)

    zv = jnp.zeros((N_VERTS, 5), f32)
    tvv = jnp.concatenate([verts, zv, vnormals, zv], axis=1)

    chem_out, tp = _chem_stage(graph_x, node_pos, W_chem1, bc1, W_chem2, bc2, wp)

    yg, stg1 = _lin1_stage(surface_x, W_geom1, bg1)
    stg2 = _mid_stage(yg, stg1, W_geom2, bg2)
    hg = _apply_stage(yg, stg1, stg2, W_geom2, bg2)

    gath_p, gath_vv = _gather_sc(tp, tvv, ind, vids)

    dist2d = vert_nbr_dist.reshape(E, 1)
    y, st1 = _edge_a(gath_p, gath_vv, dist2d, w1da, b1)
    st2 = _mid_stage(y, st1, W_surf2, b2)
    u = _edge_c(y, st1, st2, w2f, b2f, w2c, b2c)

    agg_full = _scatter_sc(u, vids)
    agg = agg_full[0:N_VERTS]

    yf, stf1 = _fin_a(agg, hg, wf1a, wf1b, bf1)
    stf2 = _mid_stage(yf, stf1, W_feat2, bf2)
    h_geom = _apply_stage(yf, stf1, stf2, W_feat2, bf2)

    return (h_geom, chem_out)
